# Initial kernel scaffold; baseline (speedup 1.0000x reference)
import jax
import jax.numpy as jnp
from jax.experimental import pallas as pl


def kernel(feature_buffer, resid_acts, W_enc, W_dec, b_enc, b_dec, up_W_dec, up_b_dec, up_feat_acts, conn_mask):
    raise NotImplementedError("write your pallas kernel here")


# speedup vs baseline: 3.3165x; 3.3165x over previous
import functools

import jax
import jax.numpy as jnp
from jax import lax
from jax.experimental import pallas as pl

D_MODEL = 768
DICT = 24576
K = 128
TT = 64


def _sortable_u32(x):
    u = lax.bitcast_convert_type(x, jnp.uint32)
    neg = (u >> 31).astype(jnp.bool_)
    flip = jnp.where(neg, jnp.uint32(0xFFFFFFFF), jnp.uint32(0x80000000))
    return u ^ flip


def _select_body(acts_ref, out_ref):
    s = _sortable_u32(acts_ref[...])

    kk = jnp.int32(K)

    lo0 = jnp.zeros((TT, 1), jnp.uint32)
    hi0 = jnp.full((TT, 1), 0xFFFFFFFF, jnp.uint32)

    def bs_body(_, lohi):
        lo, hi = lohi
        mid = lo + ((hi - lo) >> 1)
        cnt = jnp.sum((s > mid).astype(jnp.int32), axis=1, keepdims=True)
        ge = cnt >= kk
        lo2 = jnp.where(ge, mid, lo)
        hi2 = jnp.where(ge, hi, mid)
        return lo2, hi2

    lo, hi = lax.fori_loop(0, 32, bs_body, (lo0, hi0))
    tau = hi

    n_gt = jnp.sum((s > tau).astype(jnp.int32), axis=1, keepdims=True)
    r = kk - n_gt

    eq = s == tau
    idx = lax.broadcasted_iota(jnp.int32, (TT, DICT), 1)

    clo0 = jnp.zeros((TT, 1), jnp.int32)
    chi0 = jnp.full((TT, 1), DICT, jnp.int32)

    def cs_body(_, lohi):
        lo_, hi_ = lohi
        mid = lo_ + ((hi_ - lo_) >> 1)
        m = jnp.sum((eq & (idx < mid)).astype(jnp.int32), axis=1, keepdims=True)
        lt = m < r
        return jnp.where(lt, mid, lo_), jnp.where(lt, hi_, mid)

    _, cstar = lax.fori_loop(0, 15, cs_body, (clo0, chi0))

    sel = ((s > tau) | (eq & (idx < cstar))) & (s > jnp.uint32(0x80000000))
    xpos = lax.bitcast_convert_type(s ^ jnp.uint32(0x80000000), jnp.float32)
    out_ref[...] = jnp.where(sel, xpos, 0.0)


@jax.jit
def _select_topk_mask(acts):
    n = acts.shape[0]
    return pl.pallas_call(
        _select_body,
        grid=(n // TT,),
        in_specs=[pl.BlockSpec((TT, DICT), lambda i: (i, 0))],
        out_specs=pl.BlockSpec((TT, DICT), lambda i: (i, 0)),
        out_shape=jax.ShapeDtypeStruct((n, DICT), jnp.float32),
    )(acts)


def kernel(feature_buffer, resid_acts, W_enc, W_dec, b_enc, b_dec,
           up_W_dec, up_b_dec, up_feat_acts, conn_mask):
    n_up = up_W_dec.shape[0]
    approx_acts = jnp.einsum('bsd,df->bsf', resid_acts - b_dec, W_enc)
    upstream_bias = jnp.zeros((D_MODEL,), dtype=jnp.float32)
    for i in range(n_up):
        upstream_bias = upstream_bias + up_b_dec[i]
        virtual_w = jnp.einsum('ud,df->uf', up_W_dec[i], W_enc)
        virtual_w = virtual_w * conn_mask[i]
        approx_acts = approx_acts + jnp.einsum(
            'bsu,uf->bsf', up_feat_acts[i], virtual_w)
    bias_contrib = jnp.einsum('d,df->f', upstream_bias, W_enc)
    approx_acts = approx_acts + b_enc + bias_contrib

    b, sl, _ = approx_acts.shape
    masked = _select_topk_mask(approx_acts.reshape(b * sl, DICT))
    recon = jnp.einsum('nf,fd->nd', masked, W_dec).reshape(b, sl, D_MODEL)
    return recon + b_dec


# speedup vs baseline: 3.8484x; 1.1604x over previous
import functools

import jax
import jax.numpy as jnp
from jax import lax
from jax.experimental import pallas as pl
from jax.experimental.pallas import tpu as pltpu

D_MODEL = 768
DICT = 24576
K = 128
TT = 64


def _sortable_u32(x):
    u = lax.bitcast_convert_type(x, jnp.uint32)
    neg = (u >> 31).astype(jnp.bool_)
    flip = jnp.where(neg, jnp.uint32(0xFFFFFFFF), jnp.uint32(0x80000000))
    return u ^ flip


def _select_body(acts_ref, out_ref):
    s = _sortable_u32(acts_ref[...])

    kk = jnp.int32(K)

    lo0 = jnp.zeros((TT, 1), jnp.uint32)
    hi0 = jnp.full((TT, 1), 0xFFFFFFFF, jnp.uint32)

    def bs_body(_, lohi):
        lo, hi = lohi
        mid = lo + ((hi - lo) >> 1)
        cnt = jnp.sum((s > mid).astype(jnp.int32), axis=1, keepdims=True)
        ge = cnt >= kk
        lo2 = jnp.where(ge, mid, lo)
        hi2 = jnp.where(ge, hi, mid)
        return lo2, hi2

    lo, hi = lax.fori_loop(0, 32, bs_body, (lo0, hi0))
    tau = hi

    n_gt = jnp.sum((s > tau).astype(jnp.int32), axis=1, keepdims=True)
    r = kk - n_gt

    eq = s == tau
    idx = lax.broadcasted_iota(jnp.int32, (TT, DICT), 1)

    clo0 = jnp.zeros((TT, 1), jnp.int32)
    chi0 = jnp.full((TT, 1), DICT, jnp.int32)

    def cs_body(_, lohi):
        lo_, hi_ = lohi
        mid = lo_ + ((hi_ - lo_) >> 1)
        m = jnp.sum((eq & (idx < mid)).astype(jnp.int32), axis=1, keepdims=True)
        lt = m < r
        return jnp.where(lt, mid, lo_), jnp.where(lt, hi_, mid)

    _, cstar = lax.fori_loop(0, 15, cs_body, (clo0, chi0))

    sel = ((s > tau) | (eq & (idx < cstar))) & (s > jnp.uint32(0x80000000))
    xpos = lax.bitcast_convert_type(s ^ jnp.uint32(0x80000000), jnp.float32)
    out_ref[...] = jnp.where(sel, xpos, 0.0)


@jax.jit
def _select_topk_mask(acts):
    n = acts.shape[0]
    return pl.pallas_call(
        _select_body,
        grid=(n // TT,),
        in_specs=[pl.BlockSpec((TT, DICT), lambda i: (i, 0))],
        out_specs=pl.BlockSpec((TT, DICT), lambda i: (i, 0)),
        out_shape=jax.ShapeDtypeStruct((n, DICT), jnp.float32),
    )(acts)


_HI = lax.Precision.HIGHEST


def _vw_body(u_ref, we_ref, m_ref, vw_ref):
    vw = jnp.dot(u_ref[0].astype(jnp.bfloat16),
                 we_ref[...].astype(jnp.bfloat16),
                 preferred_element_type=jnp.float32)
    vw_ref[0] = (vw * m_ref[0]).astype(jnp.bfloat16)


def _masked_vw(up_W_dec, W_enc, conn_mask):
    n_up = up_W_dec.shape[0]
    bu = bf = 1024
    return pl.pallas_call(
        _vw_body,
        grid=(n_up, DICT // bf, DICT // bu),
        in_specs=[
            pl.BlockSpec((1, bu, D_MODEL), lambda i, f, u: (i, u, 0)),
            pl.BlockSpec((D_MODEL, bf), lambda i, f, u: (0, f)),
            pl.BlockSpec((1, bu, bf), lambda i, f, u: (i, u, f)),
        ],
        out_specs=pl.BlockSpec((1, bu, bf), lambda i, f, u: (i, u, f)),
        out_shape=jax.ShapeDtypeStruct((n_up, DICT, DICT), jnp.bfloat16),
    )(up_W_dec, W_enc, conn_mask)


def _pruned_body(a_ref, vw_ref, out_ref, acc_ref):
    i = pl.program_id(2)
    k = pl.program_id(3)
    ni = pl.num_programs(2)
    nk = pl.num_programs(3)

    @pl.when((i == 0) & (k == 0))
    def _():
        acc_ref[...] = jnp.zeros_like(acc_ref)

    acc_ref[...] += jnp.dot(a_ref[0], vw_ref[0],
                            preferred_element_type=jnp.float32)

    @pl.when((i == ni - 1) & (k == nk - 1))
    def _():
        out_ref[...] = acc_ref[...]


def _pruned(A_bf, vw_bf):
    n_up, n, _ = A_bf.shape
    tt, bf, bk = 1024, 2048, 2048
    return pl.pallas_call(
        _pruned_body,
        grid=(n // tt, DICT // bf, n_up, DICT // bk),
        in_specs=[
            pl.BlockSpec((1, tt, bk), lambda t, f, i, k: (i, t, k)),
            pl.BlockSpec((1, bk, bf), lambda t, f, i, k: (i, k, f)),
        ],
        out_specs=pl.BlockSpec((tt, bf), lambda t, f, i, k: (t, f)),
        out_shape=jax.ShapeDtypeStruct((n, DICT), jnp.float32),
        scratch_shapes=[pltpu.VMEM((tt, bf), jnp.float32)],
    )(A_bf, vw_bf)


def _enc_body(x_ref, we_ref, pr_ref, be_ref, out_ref):
    out_ref[...] = (
        jnp.dot(x_ref[...].astype(jnp.bfloat16),
                we_ref[...].astype(jnp.bfloat16),
                preferred_element_type=jnp.float32)
        + pr_ref[...] + be_ref[...])


def _encode_add(x, W_enc, pruned, b_enc):
    n = x.shape[0]
    tt, bf = 256, 4096
    return pl.pallas_call(
        _enc_body,
        grid=(DICT // bf, n // tt),
        in_specs=[
            pl.BlockSpec((tt, D_MODEL), lambda f, t: (t, 0)),
            pl.BlockSpec((D_MODEL, bf), lambda f, t: (0, f)),
            pl.BlockSpec((tt, bf), lambda f, t: (t, f)),
            pl.BlockSpec((1, bf), lambda f, t: (0, f)),
        ],
        out_specs=pl.BlockSpec((tt, bf), lambda f, t: (t, f)),
        out_shape=jax.ShapeDtypeStruct((n, DICT), jnp.float32),
    )(x, W_enc, pruned, b_enc.reshape(1, DICT))


def _bias_body(u_ref, w_ref, o_ref):
    o_ref[...] = jnp.dot(u_ref[...].astype(jnp.bfloat16),
                         w_ref[...].astype(jnp.bfloat16),
                         preferred_element_type=jnp.float32)


def _bias_contrib(ub, W_enc):
    bf = 4096
    return pl.pallas_call(
        _bias_body,
        grid=(DICT // bf,),
        in_specs=[pl.BlockSpec((1, D_MODEL), lambda f: (0, 0)),
                  pl.BlockSpec((D_MODEL, bf), lambda f: (0, f))],
        out_specs=pl.BlockSpec((1, bf), lambda f: (0, f)),
        out_shape=jax.ShapeDtypeStruct((1, DICT), jnp.float32),
    )(ub.reshape(1, D_MODEL), W_enc)


def _dec_body(m_ref, wd_ref, bd_ref, out_ref, acc_ref):
    k = pl.program_id(1)
    nk = pl.num_programs(1)

    @pl.when(k == 0)
    def _():
        acc_ref[...] = jnp.zeros_like(acc_ref)

    acc_ref[...] += jnp.dot(m_ref[...].astype(jnp.bfloat16),
                            wd_ref[...].astype(jnp.bfloat16),
                            preferred_element_type=jnp.float32)

    @pl.when(k == nk - 1)
    def _():
        out_ref[...] = acc_ref[...] + bd_ref[...]


def _decode(masked, W_dec, b_dec):
    n = masked.shape[0]
    tt, bk = 512, 2048
    return pl.pallas_call(
        _dec_body,
        grid=(n // tt, DICT // bk),
        in_specs=[
            pl.BlockSpec((tt, bk), lambda t, k: (t, k)),
            pl.BlockSpec((bk, D_MODEL), lambda t, k: (k, 0)),
            pl.BlockSpec((1, D_MODEL), lambda t, k: (0, 0)),
        ],
        out_specs=pl.BlockSpec((tt, D_MODEL), lambda t, k: (t, 0)),
        out_shape=jax.ShapeDtypeStruct((n, D_MODEL), jnp.float32),
        scratch_shapes=[pltpu.VMEM((tt, D_MODEL), jnp.float32)],
    )(masked, W_dec, b_dec.reshape(1, D_MODEL))


def kernel(feature_buffer, resid_acts, W_enc, W_dec, b_enc, b_dec,
           up_W_dec, up_b_dec, up_feat_acts, conn_mask):
    n_up = up_W_dec.shape[0]
    b, sl, _ = resid_acts.shape
    n = b * sl

    vw_bf = _masked_vw(up_W_dec, W_enc, conn_mask)
    A_bf = up_feat_acts.reshape(n_up, n, DICT).astype(jnp.bfloat16)
    pruned = _pruned(A_bf, vw_bf)

    ub = jnp.sum(up_b_dec, axis=0)
    x = resid_acts.reshape(n, D_MODEL) - b_dec
    brow = b_enc.reshape(1, DICT) + _bias_contrib(ub, W_enc)
    acts = _encode_add(x, W_enc, pruned, brow)
    masked = _select_topk_mask(acts)
    return _decode(masked, W_dec, b_dec).reshape(b, sl, D_MODEL)


# speedup vs baseline: 3.8497x; 1.0003x over previous
import functools

import jax
import jax.numpy as jnp
from jax import lax
from jax.experimental import pallas as pl
from jax.experimental.pallas import tpu as pltpu

D_MODEL = 768
DICT = 24576
K = 128
TT = 64


def _sortable_u32(x):
    u = lax.bitcast_convert_type(x, jnp.uint32)
    neg = (u >> 31).astype(jnp.bool_)
    flip = jnp.where(neg, jnp.uint32(0xFFFFFFFF), jnp.uint32(0x80000000))
    return u ^ flip


def _select_body(acts_ref, out_ref):
    s = _sortable_u32(acts_ref[...])

    kk = jnp.int32(K)

    lo0 = jnp.zeros((TT, 1), jnp.uint32)
    hi0 = jnp.full((TT, 1), 0xFFFFFFFF, jnp.uint32)

    def bs_body(_, lohi):
        lo, hi = lohi
        mid = lo + ((hi - lo) >> 1)
        cnt = jnp.sum((s > mid).astype(jnp.int32), axis=1, keepdims=True)
        ge = cnt >= kk
        lo2 = jnp.where(ge, mid, lo)
        hi2 = jnp.where(ge, hi, mid)
        return lo2, hi2

    lo, hi = lax.fori_loop(0, 32, bs_body, (lo0, hi0))
    tau = hi

    n_gt = jnp.sum((s > tau).astype(jnp.int32), axis=1, keepdims=True)
    r = kk - n_gt

    eq = s == tau
    idx = lax.broadcasted_iota(jnp.int32, (TT, DICT), 1)

    clo0 = jnp.zeros((TT, 1), jnp.int32)
    chi0 = jnp.full((TT, 1), DICT, jnp.int32)

    def cs_body(_, lohi):
        lo_, hi_ = lohi
        mid = lo_ + ((hi_ - lo_) >> 1)
        m = jnp.sum((eq & (idx < mid)).astype(jnp.int32), axis=1, keepdims=True)
        lt = m < r
        return jnp.where(lt, mid, lo_), jnp.where(lt, hi_, mid)

    _, cstar = lax.fori_loop(0, 15, cs_body, (clo0, chi0))

    sel = ((s > tau) | (eq & (idx < cstar))) & (s > jnp.uint32(0x80000000))
    xpos = lax.bitcast_convert_type(s ^ jnp.uint32(0x80000000), jnp.float32)
    out_ref[...] = jnp.where(sel, xpos, 0.0)


@jax.jit
def _select_topk_mask(acts):
    n = acts.shape[0]
    return pl.pallas_call(
        _select_body,
        grid=(n // TT,),
        in_specs=[pl.BlockSpec((TT, DICT), lambda i: (i, 0))],
        out_specs=pl.BlockSpec((TT, DICT), lambda i: (i, 0)),
        out_shape=jax.ShapeDtypeStruct((n, DICT), jnp.float32),
    )(acts)


def _vw_body(u_ref, we_ref, m_ref, vw_ref):
    vw = jnp.dot(u_ref[0].astype(jnp.bfloat16),
                 we_ref[...].astype(jnp.bfloat16),
                 preferred_element_type=jnp.float32)
    vw_ref[0] = (vw * m_ref[0]).astype(jnp.bfloat16)


def _masked_vw(up_W_dec, W_enc, conn_mask):
    n_up = up_W_dec.shape[0]
    bu = bf = 1024
    return pl.pallas_call(
        _vw_body,
        grid=(n_up, DICT // bf, DICT // bu),
        in_specs=[
            pl.BlockSpec((1, bu, D_MODEL), lambda i, f, u: (i, u, 0)),
            pl.BlockSpec((D_MODEL, bf), lambda i, f, u: (0, f)),
            pl.BlockSpec((1, bu, bf), lambda i, f, u: (i, u, f)),
        ],
        out_specs=pl.BlockSpec((1, bu, bf), lambda i, f, u: (i, u, f)),
        out_shape=jax.ShapeDtypeStruct((n_up, DICT, DICT), jnp.bfloat16),
    )(up_W_dec, W_enc, conn_mask)


def _pruned_body(a_ref, vw_ref, out_ref, acc_ref):
    i = pl.program_id(2)
    k = pl.program_id(3)
    ni = pl.num_programs(2)
    nk = pl.num_programs(3)

    @pl.when((i == 0) & (k == 0))
    def _():
        acc_ref[...] = jnp.zeros_like(acc_ref)

    acc_ref[...] += jnp.dot(a_ref[0], vw_ref[0],
                            preferred_element_type=jnp.float32)

    @pl.when((i == ni - 1) & (k == nk - 1))
    def _():
        out_ref[...] = acc_ref[...]


def _pruned(A_bf, vw_bf):
    n_up, n, _ = A_bf.shape
    tt, bf, bk = 1024, 2048, 2048
    return pl.pallas_call(
        _pruned_body,
        grid=(n // tt, DICT // bf, n_up, DICT // bk),
        in_specs=[
            pl.BlockSpec((1, tt, bk), lambda t, f, i, k: (i, t, k)),
            pl.BlockSpec((1, bk, bf), lambda t, f, i, k: (i, k, f)),
        ],
        out_specs=pl.BlockSpec((tt, bf), lambda t, f, i, k: (t, f)),
        out_shape=jax.ShapeDtypeStruct((n, DICT), jnp.float32),
        scratch_shapes=[pltpu.VMEM((tt, bf), jnp.float32)],
    )(A_bf, vw_bf)


def _enc_body(x_ref, we_ref, pr_ref, be_ref, out_ref):
    out_ref[...] = (
        jnp.dot(x_ref[...].astype(jnp.bfloat16),
                we_ref[...].astype(jnp.bfloat16),
                preferred_element_type=jnp.float32)
        + pr_ref[...] + be_ref[...])


def _encode_add(x, W_enc, pruned, b_enc):
    n = x.shape[0]
    tt, bf = 256, 4096
    return pl.pallas_call(
        _enc_body,
        grid=(DICT // bf, n // tt),
        in_specs=[
            pl.BlockSpec((tt, D_MODEL), lambda f, t: (t, 0)),
            pl.BlockSpec((D_MODEL, bf), lambda f, t: (0, f)),
            pl.BlockSpec((tt, bf), lambda f, t: (t, f)),
            pl.BlockSpec((1, bf), lambda f, t: (0, f)),
        ],
        out_specs=pl.BlockSpec((tt, bf), lambda f, t: (t, f)),
        out_shape=jax.ShapeDtypeStruct((n, DICT), jnp.float32),
    )(x, W_enc, pruned, b_enc.reshape(1, DICT))


def _bias_body(u_ref, w_ref, o_ref):
    o_ref[...] = jnp.dot(u_ref[...].astype(jnp.bfloat16),
                         w_ref[...].astype(jnp.bfloat16),
                         preferred_element_type=jnp.float32)


def _bias_contrib(ub, W_enc):
    bf = 4096
    return pl.pallas_call(
        _bias_body,
        grid=(DICT // bf,),
        in_specs=[pl.BlockSpec((1, D_MODEL), lambda f: (0, 0)),
                  pl.BlockSpec((D_MODEL, bf), lambda f: (0, f))],
        out_specs=pl.BlockSpec((1, bf), lambda f: (0, f)),
        out_shape=jax.ShapeDtypeStruct((1, DICT), jnp.float32),
    )(ub.reshape(1, D_MODEL), W_enc)


def _dec_body(m_ref, wd_ref, bd_ref, out_ref, acc_ref):
    k = pl.program_id(1)
    nk = pl.num_programs(1)

    @pl.when(k == 0)
    def _():
        acc_ref[...] = jnp.zeros_like(acc_ref)

    acc_ref[...] += jnp.dot(m_ref[...].astype(jnp.bfloat16),
                            wd_ref[...].astype(jnp.bfloat16),
                            preferred_element_type=jnp.float32)

    @pl.when(k == nk - 1)
    def _():
        out_ref[...] = acc_ref[...] + bd_ref[...]


def _decode(masked, W_dec, b_dec):
    n = masked.shape[0]
    tt, bk = 512, 2048
    return pl.pallas_call(
        _dec_body,
        grid=(n // tt, DICT // bk),
        in_specs=[
            pl.BlockSpec((tt, bk), lambda t, k: (t, k)),
            pl.BlockSpec((bk, D_MODEL), lambda t, k: (k, 0)),
            pl.BlockSpec((1, D_MODEL), lambda t, k: (0, 0)),
        ],
        out_specs=pl.BlockSpec((tt, D_MODEL), lambda t, k: (t, 0)),
        out_shape=jax.ShapeDtypeStruct((n, D_MODEL), jnp.float32),
        scratch_shapes=[pltpu.VMEM((tt, D_MODEL), jnp.float32)],
    )(masked, W_dec, b_dec.reshape(1, D_MODEL))


def kernel(feature_buffer, resid_acts, W_enc, W_dec, b_enc, b_dec,
           up_W_dec, up_b_dec, up_feat_acts, conn_mask):
    n_up = up_W_dec.shape[0]
    b, sl, _ = resid_acts.shape
    n = b * sl

    vw_bf = _masked_vw(up_W_dec, W_enc, conn_mask)
    A_bf = up_feat_acts.reshape(n_up, n, DICT).astype(jnp.bfloat16)
    pruned = _pruned(A_bf, vw_bf)

    ub = jnp.sum(up_b_dec, axis=0)
    x = resid_acts.reshape(n, D_MODEL) - b_dec
    brow = b_enc.reshape(1, DICT) + _bias_contrib(ub, W_enc)
    acts = _encode_add(x, W_enc, pruned, brow)
    masked = _select_topk_mask(acts)
    return _decode(masked, W_dec, b_dec).reshape(b, sl, D_MODEL)
